# Initial kernel scaffold; baseline (speedup 1.0000x reference)
import jax
import jax.numpy as jnp
from jax.experimental import pallas as pl


def kernel(x, pe, edge_index, batch, atom_emb_w, pe_w1, pe_b1, pe_w2, pe_b2, pe_bn_g, pe_bn_b, in_w, in_b, conv_w1, conv_b1, conv_w2, conv_b2, bn_g, bn_b, ro_w1, ro_b1, ro_w2, ro_b2):
    raise NotImplementedError("write your pallas kernel here")


# speedup vs baseline: 4.5510x; 4.5510x over previous
import functools

import jax
import jax.numpy as jnp
from jax import lax
from jax.experimental import pallas as pl
from jax.experimental.pallas import tpu as pltpu
from jax.experimental.pallas import tpu_sc as plsc

N = 10000
E = 320000
G = 512
H = 128
L = 4
NUM_ATOM = 28
ATOM_EMB = 64
NPAD = 10240
NCORE = 2
NSUB = 16
TILES = NCORE * NSUB
CHUNK = 128
CPT = -(-E // (TILES * CHUNK))
EPAD = TILES * CPT * CHUNK
GPAD = 640
RPT = NPAD // TILES
PCH = 80

f32 = jnp.float32

_mesh = plsc.VectorSubcoreMesh(core_axis_name="c", subcore_axis_name="s")


def _zero_vmem(ref, nrows):
    z16 = jnp.zeros((1, 16), f32)

    @pl.loop(0, nrows)
    def _(r):
        @pl.loop(0, H, step=16)
        def _(k):
            ref.at[pl.ds(r, 1), pl.ds(k, 16)][...] = z16


@functools.partial(
    pl.kernel,
    out_type=jax.ShapeDtypeStruct((NCORE, NPAD, H), f32),
    mesh=_mesh,
    scratch_types=[
        pltpu.VMEM((CPT, CHUNK), jnp.int32),
        pltpu.VMEM((CPT, CHUNK), jnp.int32),
        pltpu.VMEM((CHUNK, H), f32),
        pltpu.VMEM_SHARED((NPAD, H), f32),
        pltpu.SemaphoreType.DMA,
    ],
)
def _agg(h_hbm, src_hbm, dst_hbm, out_hbm, src_v, dst_v, rows_v, acc, sem):
    c = lax.axis_index("c")
    s = lax.axis_index("s")
    tid = c * NSUB + s
    rpt = NPAD // NSUB

    _zero_vmem(rows_v, CHUNK)

    @pl.loop(0, rpt // CHUNK)
    def _(k):
        pltpu.sync_copy(rows_v, acc.at[pl.ds(s * rpt + k * CHUNK, CHUNK)])

    pltpu.sync_copy(src_hbm.at[tid], src_v)
    pltpu.sync_copy(dst_hbm.at[tid], dst_v)

    plsc.subcore_barrier()

    @pl.loop(0, CPT)
    def _(j):
        pltpu.async_copy(h_hbm.at[src_v.at[j]], rows_v, sem).wait()
        pltpu.sync_copy(rows_v, acc.at[dst_v.at[j]], add=True)

    plsc.subcore_barrier()

    pltpu.sync_copy(acc.at[pl.ds(s * rpt, rpt)], out_hbm.at[c, pl.ds(s * rpt, rpt)])


@functools.partial(
    pl.kernel,
    out_type=jax.ShapeDtypeStruct((NCORE, GPAD, H), f32),
    mesh=_mesh,
    scratch_types=[
        pltpu.VMEM((RPT // PCH, PCH), jnp.int32),
        pltpu.VMEM((PCH, H), f32),
        pltpu.VMEM_SHARED((GPAD, H), f32),
        pltpu.SemaphoreType.DMA,
    ],
)
def _pool(h_hbm, b_hbm, out_hbm, b_v, rows_v, acc, sem):
    c = lax.axis_index("c")
    s = lax.axis_index("s")
    tid = c * NSUB + s
    gpt = GPAD // NSUB

    _zero_vmem(rows_v, PCH)
    pltpu.sync_copy(rows_v.at[pl.ds(0, gpt)], acc.at[pl.ds(s * gpt, gpt)])
    pltpu.sync_copy(b_hbm.at[tid], b_v)

    plsc.subcore_barrier()

    @pl.loop(0, RPT // PCH)
    def _(j):
        pltpu.sync_copy(h_hbm.at[pl.ds(tid * RPT + j * PCH, PCH)], rows_v)
        pltpu.sync_copy(rows_v, acc.at[b_v.at[j]], add=True)

    plsc.subcore_barrier()

    pltpu.sync_copy(acc.at[pl.ds(s * gpt, gpt)], out_hbm.at[c, pl.ds(s * gpt, gpt)])


def _dot(a, b):
    return lax.dot_general(
        a, b, (((1,), (0,)), ((), ())),
        precision=lax.Precision.DEFAULT,
        preferred_element_type=f32,
    )


def _bn_relu(z, g, b, relu):
    m = jnp.mean(z, axis=0, keepdims=True)
    v = jnp.mean((z - m) ** 2, axis=0, keepdims=True)
    out = (z - m) * lax.rsqrt(v + 1e-5) * g + b
    return jnp.maximum(out, 0.0) if relu else out


def _prologue_body(x_ref, pe_ref, aw_ref, pw1_ref, pb1_ref, pw2_ref, pb2_ref,
                   pg_ref, pbb_ref, inw_ref, inb_ref, out_ref):
    xv = x_ref[...]
    oh = (xv == lax.broadcasted_iota(jnp.int32, (1, NUM_ATOM), 1)).astype(f32)
    h_atom = _dot(oh, aw_ref[...])
    t = jnp.maximum(_dot(pe_ref[...], pw1_ref[...]) + pb1_ref[...], 0.0)
    hpe = _dot(t, pw2_ref[...]) + pb2_ref[...]
    hpe = _bn_relu(hpe, pg_ref[...], pbb_ref[...], relu=False)
    h = (_dot(h_atom, inw_ref[0:ATOM_EMB, :])
         + _dot(hpe, inw_ref[ATOM_EMB:, :]) + inb_ref[...])
    out_ref[0:N, :] = h
    out_ref[N:NPAD, :] = jnp.zeros((NPAD - N, H), f32)


def _layer_body(h_ref, p_ref, w1_ref, b1_ref, w2_ref, b2_ref, g_ref, bb_ref,
                out_ref):
    z = h_ref[0:N, :] + p_ref[0, 0:N, :] + p_ref[1, 0:N, :]
    a = jnp.maximum(_dot(z, w1_ref[...]) + b1_ref[...], 0.0)
    z2 = _dot(a, w2_ref[...]) + b2_ref[...]
    out_ref[0:N, :] = _bn_relu(z2, g_ref[...], bb_ref[...], relu=True)
    out_ref[N:NPAD, :] = jnp.zeros((NPAD - N, H), f32)


def _readout_body(p_ref, w1_ref, b1_ref, w2_ref, b2_ref, out_ref):
    g = p_ref[0, 0:G, :] + p_ref[1, 0:G, :]
    a = jnp.maximum(_dot(g, w1_ref[...]) + b1_ref[...], 0.0)
    out_ref[...] = _dot(a, w2_ref[...]) + b2_ref[...]


def kernel(x, pe, edge_index, batch, atom_emb_w, pe_w1, pe_b1, pe_w2, pe_b2,
           pe_bn_g, pe_bn_b, in_w, in_b, conv_w1, conv_b1, conv_w2, conv_b2,
           bn_g, bn_b, ro_w1, ro_b1, ro_w2, ro_b2):
    x2 = x.astype(jnp.int32).reshape(N, 1)
    src = edge_index[0].astype(jnp.int32)
    dst = edge_index[1].astype(jnp.int32)
    pad_e = EPAD - E
    pad_src = jnp.zeros((pad_e,), jnp.int32)
    pad_dst = N + (jnp.arange(pad_e, dtype=jnp.int32) % (NPAD - N))
    src_slab = jnp.concatenate([src, pad_src]).reshape(TILES, CPT, CHUNK)
    dst_slab = jnp.concatenate([dst, pad_dst]).reshape(TILES, CPT, CHUNK)
    bpad = G + (jnp.arange(NPAD - N, dtype=jnp.int32) % (GPAD - G))
    batch_slab = jnp.concatenate([batch.astype(jnp.int32), bpad]).reshape(
        TILES, RPT // PCH, PCH)

    h = pl.pallas_call(
        _prologue_body,
        out_shape=jax.ShapeDtypeStruct((NPAD, H), f32),
    )(x2, pe, atom_emb_w, pe_w1, pe_b1.reshape(1, -1), pe_w2,
      pe_b2.reshape(1, -1), pe_bn_g.reshape(1, -1), pe_bn_b.reshape(1, -1),
      in_w, in_b.reshape(1, -1))

    layer = pl.pallas_call(
        _layer_body,
        out_shape=jax.ShapeDtypeStruct((NPAD, H), f32),
    )
    for i in range(L):
        parts = _agg(h, src_slab, dst_slab)
        h = layer(h, parts, conv_w1[i], conv_b1[i].reshape(1, -1), conv_w2[i],
                  conv_b2[i].reshape(1, -1), bn_g[i].reshape(1, -1),
                  bn_b[i].reshape(1, -1))

    pool = _pool(h, batch_slab)
    out = pl.pallas_call(
        _readout_body,
        out_shape=jax.ShapeDtypeStruct((G, 1), f32),
    )(pool, ro_w1, ro_b1.reshape(1, -1), ro_w2, ro_b2.reshape(1, 1))
    return out[:, 0]


# speedup vs baseline: 4.8357x; 1.0626x over previous
import functools

import jax
import jax.numpy as jnp
from jax import lax
from jax.experimental import pallas as pl
from jax.experimental.pallas import tpu as pltpu
from jax.experimental.pallas import tpu_sc as plsc

N = 10000
E = 320000
G = 512
H = 128
L = 4
NUM_ATOM = 28
ATOM_EMB = 64
NPAD = 10240
NCORE = 2
NSUB = 16
TILES = NCORE * NSUB
CHUNK = 64
NBUF = 2
SLOTS = 8
EPT = E // TILES
CPT = 160
CPT_ALL = CPT + SLOTS
GPAD = 640
RPT = NPAD // TILES
PCH = 80

f32 = jnp.float32

_mesh = plsc.VectorSubcoreMesh(core_axis_name="c", subcore_axis_name="s")


def _zero_vmem(ref, nrows):
    z16 = jnp.zeros((1, 16), f32)

    @pl.loop(0, nrows)
    def _(r):
        @pl.loop(0, H, step=16)
        def _(k):
            ref.at[pl.ds(r, 1), pl.ds(k, 16)][...] = z16


@functools.partial(
    pl.kernel,
    out_type=jax.ShapeDtypeStruct((NCORE, NPAD, H), f32),
    mesh=_mesh,
    scratch_types=[
        [pltpu.VMEM((CHUNK, H), f32) for _ in range(NBUF)],
        [pltpu.VMEM((2, CHUNK), jnp.int32) for _ in range(SLOTS)],
        pltpu.VMEM_SHARED((NPAD, H), f32),
        [pltpu.SemaphoreType.DMA for _ in range(NBUF)],
        [pltpu.SemaphoreType.DMA for _ in range(NBUF)],
        [pltpu.SemaphoreType.DMA for _ in range(SLOTS)],
    ],
)
def _agg(h_hbm, islab_hbm, out_hbm, rows, idx, acc, gsem, ssem, isem):
    c = lax.axis_index("c")
    s = lax.axis_index("s")
    tid = c * NSUB + s
    base = tid * CPT_ALL
    rpt = NPAD // NSUB

    _zero_vmem(rows[0], CHUNK)

    @pl.loop(0, rpt // CHUNK)
    def _(k):
        pltpu.sync_copy(rows[0], acc.at[pl.ds(s * rpt + k * CHUNK, CHUNK)])

    for q in range(SLOTS):
        pltpu.async_copy(islab_hbm.at[base + q], idx[q], isem[q])

    plsc.subcore_barrier()

    for b in range(NBUF):
        pltpu.make_async_copy(islab_hbm.at[base + b], idx[b], isem[b]).wait()
        pltpu.async_copy(h_hbm.at[idx[b].at[0]], rows[b], gsem[b])

    @pl.loop(0, CPT, step=SLOTS)
    def _(j):
        for q in range(SLOTS // NBUF):
            for b in range(NBUF):
                mm = NBUF * q + b
                pltpu.make_async_copy(h_hbm.at[idx[mm].at[0]], rows[b],
                                      gsem[b]).wait()
                pltpu.async_copy(rows[b], acc.at[idx[mm].at[1]], ssem[b],
                                 add=True)
            for b in range(NBUF):
                mm = NBUF * q + b
                nxt = (mm + NBUF) % SLOTS
                pltpu.make_async_copy(rows[b], acc.at[idx[mm].at[1]],
                                      ssem[b]).wait()
                pltpu.async_copy(islab_hbm.at[base + j + mm + SLOTS],
                                 idx[mm], isem[mm])
                pltpu.make_async_copy(islab_hbm.at[base + j + mm + NBUF],
                                      idx[nxt], isem[nxt]).wait()
                pltpu.async_copy(h_hbm.at[idx[nxt].at[0]], rows[b], gsem[b])

    for b in range(NBUF):
        pltpu.make_async_copy(h_hbm.at[idx[(CPT + b) % SLOTS].at[0]], rows[b],
                              gsem[b]).wait()
    for q in range(NBUF, SLOTS):
        pltpu.make_async_copy(islab_hbm.at[base + CPT + q], idx[q],
                              isem[q]).wait()

    plsc.subcore_barrier()

    pltpu.sync_copy(acc.at[pl.ds(s * rpt, rpt)],
                    out_hbm.at[c, pl.ds(s * rpt, rpt)])


@functools.partial(
    pl.kernel,
    out_type=jax.ShapeDtypeStruct((NCORE, GPAD, H), f32),
    mesh=_mesh,
    scratch_types=[
        pltpu.VMEM((RPT // PCH, PCH), jnp.int32),
        pltpu.VMEM((PCH, H), f32),
        pltpu.VMEM_SHARED((GPAD, H), f32),
        pltpu.SemaphoreType.DMA,
    ],
)
def _pool(h_hbm, b_hbm, out_hbm, b_v, rows_v, acc, sem):
    c = lax.axis_index("c")
    s = lax.axis_index("s")
    tid = c * NSUB + s
    gpt = GPAD // NSUB

    _zero_vmem(rows_v, PCH)
    pltpu.sync_copy(rows_v.at[pl.ds(0, gpt)], acc.at[pl.ds(s * gpt, gpt)])
    pltpu.sync_copy(b_hbm.at[tid], b_v)

    plsc.subcore_barrier()

    @pl.loop(0, RPT // PCH)
    def _(j):
        pltpu.sync_copy(h_hbm.at[pl.ds(tid * RPT + j * PCH, PCH)], rows_v)
        pltpu.sync_copy(rows_v, acc.at[b_v.at[j]], add=True)

    plsc.subcore_barrier()

    pltpu.sync_copy(acc.at[pl.ds(s * gpt, gpt)], out_hbm.at[c, pl.ds(s * gpt, gpt)])


def _dot(a, b):
    return lax.dot_general(
        a.astype(jnp.bfloat16), b.astype(jnp.bfloat16),
        (((1,), (0,)), ((), ())),
        precision=lax.Precision.DEFAULT,
        preferred_element_type=f32,
    )


def _bn_relu(z, g, b, relu):
    m = jnp.mean(z, axis=0, keepdims=True)
    v = jnp.mean((z - m) ** 2, axis=0, keepdims=True)
    out = (z - m) * lax.rsqrt(v + 1e-5) * g + b
    return jnp.maximum(out, 0.0) if relu else out


def _prologue_body(x_ref, pe_ref, aw_ref, pw1_ref, pb1_ref, pw2_ref, pb2_ref,
                   pg_ref, pbb_ref, inw_ref, inb_ref, out_ref):
    xv = x_ref[...]
    oh = (xv == lax.broadcasted_iota(jnp.int32, (1, NUM_ATOM), 1)).astype(f32)
    h_atom = _dot(oh, aw_ref[...])
    t = jnp.maximum(_dot(pe_ref[...], pw1_ref[...]) + pb1_ref[...], 0.0)
    hpe = _dot(t, pw2_ref[...]) + pb2_ref[...]
    hpe = _bn_relu(hpe, pg_ref[...], pbb_ref[...], relu=False)
    h = (_dot(h_atom, inw_ref[0:ATOM_EMB, :])
         + _dot(hpe, inw_ref[ATOM_EMB:, :]) + inb_ref[...])
    out_ref[0:N, :] = h
    out_ref[N:NPAD, :] = jnp.zeros((NPAD - N, H), f32)


def _layer_body(h_ref, p_ref, w1_ref, b1_ref, w2_ref, b2_ref, g_ref, bb_ref,
                out_ref):
    z = h_ref[0:N, :] + p_ref[0, 0:N, :] + p_ref[1, 0:N, :]
    a = jnp.maximum(_dot(z, w1_ref[...]) + b1_ref[...], 0.0)
    z2 = _dot(a, w2_ref[...]) + b2_ref[...]
    out_ref[0:N, :] = _bn_relu(z2, g_ref[...], bb_ref[...], relu=True)
    out_ref[N:NPAD, :] = jnp.zeros((NPAD - N, H), f32)


def _readout_body(p_ref, w1_ref, b1_ref, w2_ref, b2_ref, out_ref):
    g = p_ref[0, 0:G, :] + p_ref[1, 0:G, :]
    a = jnp.maximum(_dot(g, w1_ref[...]) + b1_ref[...], 0.0)
    out_ref[...] = _dot(a, w2_ref[...]) + b2_ref[...]


def kernel(x, pe, edge_index, batch, atom_emb_w, pe_w1, pe_b1, pe_w2, pe_b2,
           pe_bn_g, pe_bn_b, in_w, in_b, conv_w1, conv_b1, conv_w2, conv_b2,
           bn_g, bn_b, ro_w1, ro_b1, ro_w2, ro_b2):
    x2 = x.astype(jnp.int32).reshape(N, 1)
    src = edge_index[0].astype(jnp.int32)
    dst = edge_index[1].astype(jnp.int32)

    pad_e = CPT * CHUNK - EPT
    pad_src = jnp.broadcast_to(
        jnp.arange(pad_e, dtype=jnp.int32) % N, (TILES, pad_e))
    pad_dst = jnp.broadcast_to(
        N + (jnp.arange(pad_e, dtype=jnp.int32) % (NPAD - N)), (TILES, pad_e))
    src_slab = jnp.concatenate(
        [src.reshape(TILES, EPT), pad_src], axis=1).reshape(TILES, CPT, CHUNK)
    dst_slab = jnp.concatenate(
        [dst.reshape(TILES, EPT), pad_dst], axis=1).reshape(TILES, CPT, CHUNK)
    src_slab = jnp.concatenate(
        [src_slab, jnp.zeros((TILES, SLOTS, CHUNK), jnp.int32)], axis=1)
    dst_slab = jnp.concatenate(
        [dst_slab, jnp.full((TILES, SLOTS, CHUNK), N, jnp.int32)], axis=1)
    islab = jnp.stack([src_slab, dst_slab], axis=2).reshape(
        TILES * CPT_ALL, 2, CHUNK)

    bpad = G + (jnp.arange(NPAD - N, dtype=jnp.int32) % (GPAD - G))
    batch_slab = jnp.concatenate([batch.astype(jnp.int32), bpad]).reshape(
        TILES, RPT // PCH, PCH)

    h = pl.pallas_call(
        _prologue_body,
        out_shape=jax.ShapeDtypeStruct((NPAD, H), f32),
    )(x2, pe, atom_emb_w, pe_w1, pe_b1.reshape(1, -1), pe_w2,
      pe_b2.reshape(1, -1), pe_bn_g.reshape(1, -1), pe_bn_b.reshape(1, -1),
      in_w, in_b.reshape(1, -1))

    layer = pl.pallas_call(
        _layer_body,
        out_shape=jax.ShapeDtypeStruct((NPAD, H), f32),
    )
    for i in range(L):
        parts = _agg(h, islab)
        h = layer(h, parts, conv_w1[i], conv_b1[i].reshape(1, -1), conv_w2[i],
                  conv_b2[i].reshape(1, -1), bn_g[i].reshape(1, -1),
                  bn_b[i].reshape(1, -1))

    pool = _pool(h, batch_slab)
    out = pl.pallas_call(
        _readout_body,
        out_shape=jax.ShapeDtypeStruct((G, 1), f32),
    )(pool, ro_w1, ro_b1.reshape(1, -1), ro_w2, ro_b2.reshape(1, 1))
    return out[:, 0]
